# baseline (speedup 1.0000x reference)
import functools

import jax
import jax.numpy as jnp
from jax import lax
from jax.experimental import pallas as pl
from jax.experimental.pallas import tpu as pltpu
from jax.experimental.pallas import tpu_sc as plsc

_LANES = 16


@functools.lru_cache(maxsize=None)
def _make_kernel(B, S, V, D, P):
    info = plsc.get_sparse_core_info()
    NC, NS = info.num_cores, info.num_subcores
    NW = NC * NS
    N = B * S
    n_per_w = N // NW
    CHUNK = 32
    NB = 2
    NPOS = 16
    nch = n_per_w // CHUNK
    assert N % NW == 0
    assert n_per_w % CHUNK == 0
    assert nch % NB == 0
    assert B % CHUNK == 0
    assert D % _LANES == 0

    mesh = plsc.VectorSubcoreMesh(core_axis_name="c", subcore_axis_name="s")

    @functools.partial(
        pl.kernel,
        mesh=mesh,
        out_type=jax.ShapeDtypeStruct((S, B, D), jnp.float32),
        scratch_types=(
            [
                pltpu.VMEM((n_per_w,), jnp.int32),
                pltpu.VMEM((NPOS, D), jnp.float32),
            ]
            + [pltpu.VMEM((CHUNK, D), jnp.float32) for _ in range(2 * NB)]
            + [pltpu.SemaphoreType.DMA for _ in range(2 * NB)]
        ),
    )
    def emb_kernel(ids_hbm, table_hbm, pos_hbm, out_hbm, idx_v, pos_v, *rest):
        ibufs = rest[:NB]
        obufs = rest[NB : 2 * NB]
        gsems = rest[2 * NB : 3 * NB]
        ssems = rest[3 * NB : 4 * NB]

        wid = lax.axis_index("s") * NC + lax.axis_index("c")
        base = wid * n_per_w
        p_lo = (base // B) // 8 * 8

        pltpu.sync_copy(ids_hbm.at[pl.ds(base, n_per_w)], idx_v)
        pltpu.sync_copy(pos_hbm.at[pl.ds(p_lo, NPOS)], pos_v)

        def start_gather(c, b):
            pltpu.async_copy(
                table_hbm.at[idx_v.at[pl.ds(c * CHUNK, CHUNK)]], ibufs[b], gsems[b]
            )

        def wait_gather(b):
            pltpu.make_async_copy(
                table_hbm.at[pl.ds(0, CHUNK)], ibufs[b], gsems[b]
            ).wait()

        def start_scatter(c, b):
            row0 = base + c * CHUNK
            pltpu.async_copy(
                obufs[b],
                out_hbm.at[row0 // B, pl.ds(lax.rem(row0, B), CHUNK)],
                ssems[b],
            )

        def wait_scatter(b):
            pltpu.make_async_copy(
                obufs[b], out_hbm.at[0, pl.ds(0, CHUNK)], ssems[b]
            ).wait()

        def add_pos(c, b):
            src = ibufs[b]
            dst = obufs[b]
            p = (base + c * CHUNK) // B - p_lo

            @plsc.parallel_loop(0, D, _LANES, unroll=4)
            def _(off):
                sl = pl.ds(off, _LANES)
                pv = pos_v[p, sl]
                for r in range(CHUNK):
                    dst[r, sl] = src[r, sl] + pv

        for b in range(NB):
            start_gather(b, b)

        def slot(c, b):
            wait_gather(b)

            @pl.when(c >= NB)
            def _():
                wait_scatter(b)

            add_pos(c, b)
            start_scatter(c, b)

            @pl.when(c + NB < nch)
            def _():
                start_gather(c + NB, b)

        def round_body(i, carry):
            for b in range(NB):
                slot(i * NB + b, b)
            return carry

        lax.fori_loop(0, nch // NB, round_body, 0)
        for b in range(NB):
            wait_scatter(b)

    return emb_kernel


def kernel(input_ids, token_table, pos_table):
    B, S = input_ids.shape
    V, D = token_table.shape
    P = pos_table.shape[0]
    ids_smajor = input_ids.T.reshape(B * S).astype(jnp.int32)
    P_pad = ((S - 1) // 8 * 8) + 16
    pos_padded = jnp.zeros((P_pad, D), jnp.float32).at[:P].set(pos_table)
    out = _make_kernel(B, S, V, D, P)(ids_smajor, token_table, pos_padded)
    return out.transpose(1, 0, 2)


# speedup vs baseline: 1.0111x; 1.0111x over previous
import functools

import jax
import jax.numpy as jnp
from jax import lax
from jax.experimental import pallas as pl
from jax.experimental.pallas import tpu as pltpu
from jax.experimental.pallas import tpu_sc as plsc

_LANES = 16


@functools.lru_cache(maxsize=None)
def _make_kernel(B, S, V, D, P):
    info = plsc.get_sparse_core_info()
    NC, NS = info.num_cores, info.num_subcores
    NW = NC * NS
    N = B * S
    n_per_w = N // NW
    CHUNK = 16
    NB = 4
    NPOS = 16
    nch = n_per_w // CHUNK
    assert N % NW == 0
    assert n_per_w % CHUNK == 0
    assert nch % NB == 0
    assert B % CHUNK == 0
    assert D % _LANES == 0

    mesh = plsc.VectorSubcoreMesh(core_axis_name="c", subcore_axis_name="s")

    @functools.partial(
        pl.kernel,
        mesh=mesh,
        out_type=jax.ShapeDtypeStruct((S, B, D), jnp.float32),
        scratch_types=(
            [
                pltpu.VMEM((n_per_w,), jnp.int32),
                pltpu.VMEM((NPOS, D), jnp.float32),
            ]
            + [pltpu.VMEM((CHUNK, D), jnp.float32) for _ in range(2 * NB)]
            + [pltpu.SemaphoreType.DMA for _ in range(2 * NB)]
        ),
    )
    def emb_kernel(ids_hbm, table_hbm, pos_hbm, out_hbm, idx_v, pos_v, *rest):
        ibufs = rest[:NB]
        obufs = rest[NB : 2 * NB]
        gsems = rest[2 * NB : 3 * NB]
        ssems = rest[3 * NB : 4 * NB]

        wid = lax.axis_index("s") * NC + lax.axis_index("c")
        base = wid * n_per_w
        p_lo = (base // B) // 8 * 8

        pltpu.sync_copy(ids_hbm.at[pl.ds(base, n_per_w)], idx_v)
        pltpu.sync_copy(pos_hbm.at[pl.ds(p_lo, NPOS)], pos_v)

        def start_gather(c, b):
            pltpu.async_copy(
                table_hbm.at[idx_v.at[pl.ds(c * CHUNK, CHUNK)]], ibufs[b], gsems[b]
            )

        def wait_gather(b):
            pltpu.make_async_copy(
                table_hbm.at[pl.ds(0, CHUNK)], ibufs[b], gsems[b]
            ).wait()

        def start_scatter(c, b):
            row0 = base + c * CHUNK
            pltpu.async_copy(
                obufs[b],
                out_hbm.at[row0 // B, pl.ds(lax.rem(row0, B), CHUNK)],
                ssems[b],
            )

        def wait_scatter(b):
            pltpu.make_async_copy(
                obufs[b], out_hbm.at[0, pl.ds(0, CHUNK)], ssems[b]
            ).wait()

        def add_pos(c, b):
            src = ibufs[b]
            dst = obufs[b]
            p = (base + c * CHUNK) // B - p_lo

            @plsc.parallel_loop(0, D, _LANES, unroll=8)
            def _(off):
                sl = pl.ds(off, _LANES)
                pv = pos_v[p, sl]
                for r in range(CHUNK):
                    dst[r, sl] = src[r, sl] + pv

        for b in range(NB):
            start_gather(b, b)

        def slot(c, b):
            wait_gather(b)

            @pl.when(c >= NB)
            def _():
                wait_scatter(b)

            add_pos(c, b)
            start_scatter(c, b)

            @pl.when(c + NB < nch)
            def _():
                start_gather(c + NB, b)

        def round_body(i, carry):
            for b in range(NB):
                slot(i * NB + b, b)
            return carry

        lax.fori_loop(0, nch // NB, round_body, 0)
        for b in range(NB):
            wait_scatter(b)

    return emb_kernel


def kernel(input_ids, token_table, pos_table):
    B, S = input_ids.shape
    V, D = token_table.shape
    P = pos_table.shape[0]
    ids_smajor = input_ids.T.reshape(B * S).astype(jnp.int32)
    P_pad = ((S - 1) // 8 * 8) + 16
    pos_padded = jnp.zeros((P_pad, D), jnp.float32).at[:P].set(pos_table)
    out = _make_kernel(B, S, V, D, P)(ids_smajor, token_table, pos_padded)
    return out.transpose(1, 0, 2)
